# Initial kernel scaffold; baseline (speedup 1.0000x reference)
import jax
import jax.numpy as jnp
from jax.experimental import pallas as pl


def kernel(behavior_ids, category_ids, product_service_ids, source_service_ids, quantity_values, sequence_length, behavior_table, category_table, product_table, source_table, qW, qb, W_ih, W_hh, b_ih, b_hh, cW1, cb1, cW2, cb2):
    raise NotImplementedError("write your pallas kernel here")


# speedup vs baseline: 4.2121x; 4.2121x over previous
import functools

import jax
import jax.numpy as jnp
from jax import lax
from jax.experimental import pallas as pl
from jax.experimental.pallas import tpu as pltpu
from jax.experimental.pallas import tpu_sc as plsc

B, L, ED, H, NCLS = 4096, 50, 64, 256, 100
BV, CV, PV, SV = 1000, 1000, 100000, 1000


_NW = 32
_ROWS = L * B * 2
_PER_W = _ROWS // _NW
_GCH = 128
_KK = 4
_CH = _GCH * _KK
_NCHUNK = _PER_W // _CH


def _sc_gather(table_hbm, idxa_hbm, idxb_hbm, out_hbm, idxa_v, idxb_v, rows_v,
               sem):
    wid = lax.axis_index("s") * 2 + lax.axis_index("c")
    base = wid * _PER_W

    def chunk(k, _):
        off = base + k * _CH
        pltpu.sync_copy(idxa_hbm.at[pl.ds(off, _CH)], idxa_v)
        pltpu.sync_copy(idxb_hbm.at[pl.ds(off, _CH)], idxb_v)
        copies = [
            pltpu.async_copy(
                table_hbm.at[idxa_v.at[pl.ds(j * _GCH, _GCH)]],
                rows_v.at[pl.ds(j * _GCH, _GCH)],
                sem,
            )
            for j in range(_KK)
        ]
        for cpy in copies:
            cpy.wait()
        copies = [
            pltpu.async_copy(
                table_hbm.at[idxb_v.at[pl.ds(j * _GCH, _GCH)]],
                rows_v.at[pl.ds(j * _GCH, _GCH)],
                sem,
                add=True,
            )
            for j in range(_KK)
        ]
        for cpy in copies:
            cpy.wait()
        pltpu.sync_copy(rows_v, out_hbm.at[pl.ds(off, _CH)])
        return ()

    lax.fori_loop(0, _NCHUNK, chunk, (), unroll=False)


@functools.cache
def _gather_call():
    return pl.kernel(
        _sc_gather,
        out_type=jax.ShapeDtypeStruct((_ROWS, 2 * ED), jnp.float32),
        mesh=plsc.VectorSubcoreMesh(core_axis_name="c", subcore_axis_name="s"),
        scratch_types=[
            pltpu.VMEM((_CH,), jnp.int32),
            pltpu.VMEM((_CH,), jnp.int32),
            pltpu.VMEM((_CH, 2 * ED), jnp.float32),
            pltpu.SemaphoreType.DMA,
        ],
    )


_BB = 2048


def _lstm_body(emb_ref, q_ref, wx_ref, wh_ref, bias_ref, u_ref,
               w1_ref, b1_ref, w2_ref, b2_ref, out_ref, h_scr, c_scr):
    t = pl.program_id(1)

    @pl.when(t == 0)
    def _init():
        h_scr[...] = jnp.zeros_like(h_scr)
        c_scr[...] = jnp.zeros_like(c_scr)

    x = emb_ref[0]
    h = h_scr[...]
    q = q_ref[0]
    gates = (
        jnp.dot(x, wx_ref[...], preferred_element_type=jnp.float32)
        + jnp.dot(h, wh_ref[...], preferred_element_type=jnp.float32)
        + q * u_ref[...]
        + bias_ref[...]
    )
    i_g = jax.nn.sigmoid(gates[:, 0:H])
    f_g = jax.nn.sigmoid(gates[:, H:2 * H])
    g_g = jnp.tanh(gates[:, 2 * H:3 * H])
    o_g = jax.nn.sigmoid(gates[:, 3 * H:4 * H])
    c_new = f_g * c_scr[...] + i_g * g_g
    h_new = o_g * jnp.tanh(c_new)
    h_scr[...] = h_new
    c_scr[...] = c_new

    @pl.when(t == L - 1)
    def _head():
        hid = jnp.maximum(
            jnp.dot(h_new, w1_ref[...], preferred_element_type=jnp.float32)
            + b1_ref[...], 0.0)
        out_ref[...] = (
            jnp.dot(hid, w2_ref[...], preferred_element_type=jnp.float32)
            + b2_ref[...])


_lstm_call = pl.pallas_call(
    _lstm_body,
    grid=(B // _BB, L),
    in_specs=[
        pl.BlockSpec((1, _BB, 4 * ED), lambda i, t: (t, i, 0)),
        pl.BlockSpec((1, _BB, 1), lambda i, t: (t, i, 0)),
        pl.BlockSpec((4 * ED, 4 * H), lambda i, t: (0, 0)),
        pl.BlockSpec((H, 4 * H), lambda i, t: (0, 0)),
        pl.BlockSpec((1, 4 * H), lambda i, t: (0, 0)),
        pl.BlockSpec((1, 4 * H), lambda i, t: (0, 0)),
        pl.BlockSpec((H, H), lambda i, t: (0, 0)),
        pl.BlockSpec((1, H), lambda i, t: (0, 0)),
        pl.BlockSpec((H, NCLS), lambda i, t: (0, 0)),
        pl.BlockSpec((1, NCLS), lambda i, t: (0, 0)),
    ],
    out_specs=pl.BlockSpec((_BB, NCLS), lambda i, t: (i, 0)),
    out_shape=jax.ShapeDtypeStruct((B, NCLS), jnp.float32),
    scratch_shapes=[
        pltpu.VMEM((_BB, H), jnp.float32),
        pltpu.VMEM((_BB, H), jnp.float32),
    ],
    compiler_params=pltpu.CompilerParams(
        dimension_semantics=("arbitrary", "arbitrary"),
    ),
)


def kernel(behavior_ids, category_ids, product_service_ids, source_service_ids,
           quantity_values, sequence_length, behavior_table, category_table,
           product_table, source_table, qW, qb, W_ih, W_hh, b_ih, b_hh,
           cW1, cb1, cW2, cb2):
    del sequence_length

    left = jnp.concatenate([behavior_table, product_table], axis=0)
    right = jnp.concatenate([category_table, source_table], axis=0)
    z_l = jnp.zeros_like(left)
    z_r = jnp.zeros_like(right)
    table = jnp.concatenate([
        jnp.concatenate([left, z_l], axis=1),
        jnp.concatenate([z_r, right], axis=1),
    ], axis=0)
    idxa = jnp.stack(
        [behavior_ids.T.astype(jnp.int32),
         product_service_ids.T.astype(jnp.int32) + BV],
        axis=-1).reshape(_ROWS)
    idxb = jnp.stack(
        [category_ids.T.astype(jnp.int32) + (BV + PV),
         source_service_ids.T.astype(jnp.int32) + (BV + PV + CV)],
        axis=-1).reshape(_ROWS)

    emb = _gather_call()(table, idxa, idxb).reshape(L, B, 4 * ED)

    wq = W_ih[:, 4 * ED:]
    u = (wq @ qW[:, 0]).reshape(1, 4 * H)
    bias = (b_ih + b_hh + wq @ qb).reshape(1, 4 * H)
    q_t = quantity_values.T.reshape(L, B, 1)

    out = _lstm_call(
        emb, q_t,
        W_ih[:, :4 * ED].T, W_hh.T, bias, u,
        cW1.T, cb1.reshape(1, H), cW2.T, cb2.reshape(1, NCLS),
    )
    return out


# speedup vs baseline: 4.3469x; 1.0320x over previous
import functools

import jax
import jax.numpy as jnp
from jax import lax
from jax.experimental import pallas as pl
from jax.experimental.pallas import tpu as pltpu
from jax.experimental.pallas import tpu_sc as plsc

B, L, ED, H, NCLS = 4096, 50, 64, 256, 100
BV, CV, PV, SV = 1000, 1000, 100000, 1000


_NW = 32
_ROWS = L * B * 2
_PER_W = _ROWS // _NW
_GCH = 128
_KK = 4
_CH = _GCH * _KK
_NCHUNK = _PER_W // _CH


def _sc_gather(table_hbm, idxa_hbm, idxb_hbm, out_hbm, idxa_v, idxb_v, rows_v,
               sem):
    wid = lax.axis_index("s") * 2 + lax.axis_index("c")
    base = wid * _PER_W

    def chunk(k, _):
        off = base + k * _CH
        pltpu.sync_copy(idxa_hbm.at[pl.ds(off, _CH)], idxa_v)
        pltpu.sync_copy(idxb_hbm.at[pl.ds(off, _CH)], idxb_v)
        copies = [
            pltpu.async_copy(
                table_hbm.at[idxa_v.at[pl.ds(j * _GCH, _GCH)]],
                rows_v.at[pl.ds(j * _GCH, _GCH)],
                sem,
            )
            for j in range(_KK)
        ]
        for cpy in copies:
            cpy.wait()
        copies = [
            pltpu.async_copy(
                table_hbm.at[idxb_v.at[pl.ds(j * _GCH, _GCH)]],
                rows_v.at[pl.ds(j * _GCH, _GCH)],
                sem,
                add=True,
            )
            for j in range(_KK)
        ]
        for cpy in copies:
            cpy.wait()
        pltpu.sync_copy(rows_v, out_hbm.at[pl.ds(off, _CH)])
        return ()

    lax.fori_loop(0, _NCHUNK, chunk, (), unroll=False)


@functools.cache
def _gather_call():
    return pl.kernel(
        _sc_gather,
        out_type=jax.ShapeDtypeStruct((_ROWS, 2 * ED), jnp.float32),
        mesh=plsc.VectorSubcoreMesh(core_axis_name="c", subcore_axis_name="s"),
        scratch_types=[
            pltpu.VMEM((_CH,), jnp.int32),
            pltpu.VMEM((_CH,), jnp.int32),
            pltpu.VMEM((_CH, 2 * ED), jnp.float32),
            pltpu.SemaphoreType.DMA,
        ],
    )


_BB = 2048


def _sigm(x):
    return 0.5 * jnp.tanh(0.5 * x) + 0.5


def _lstm_body(emb_ref, q_ref, wx_ref, wh_ref, bias_ref, u_ref,
               w1_ref, b1_ref, w2_ref, b2_ref, out_ref, h_scr, c_scr, gx_scr):
    t = pl.program_id(1)
    first = t == 0

    gx = gx_scr[...]
    h = h_scr[...]
    gates = gx + jnp.dot(h, wh_ref[...], preferred_element_type=jnp.float32)
    i_g = _sigm(gates[:, 0:H])
    f_g = _sigm(gates[:, H:2 * H])
    g_g = jnp.tanh(gates[:, 2 * H:3 * H])
    o_g = _sigm(gates[:, 3 * H:4 * H])
    c_new = f_g * c_scr[...] + i_g * g_g
    h_new = o_g * jnp.tanh(c_new)
    h_scr[...] = jnp.where(first, 0.0, h_new)
    c_scr[...] = jnp.where(first, 0.0, c_new)

    x = emb_ref[0]
    q = q_ref[0]
    gx_scr[...] = (
        jnp.dot(x, wx_ref[...], preferred_element_type=jnp.float32)
        + q * u_ref[...] + bias_ref[...]
    )

    @pl.when(t == L)
    def _head():
        hT = h_scr[...]
        hid = jnp.maximum(
            jnp.dot(hT, w1_ref[...], preferred_element_type=jnp.float32)
            + b1_ref[...], 0.0)
        out_ref[...] = (
            jnp.dot(hid, w2_ref[...], preferred_element_type=jnp.float32)
            + b2_ref[...])


def _clamp_t(t):
    return jnp.minimum(t, L - 1)


_lstm_call = pl.pallas_call(
    _lstm_body,
    grid=(B // _BB, L + 1),
    in_specs=[
        pl.BlockSpec((1, _BB, 4 * ED), lambda i, t: (_clamp_t(t), i, 0)),
        pl.BlockSpec((1, _BB, 1), lambda i, t: (_clamp_t(t), i, 0)),
        pl.BlockSpec((4 * ED, 4 * H), lambda i, t: (0, 0)),
        pl.BlockSpec((H, 4 * H), lambda i, t: (0, 0)),
        pl.BlockSpec((1, 4 * H), lambda i, t: (0, 0)),
        pl.BlockSpec((1, 4 * H), lambda i, t: (0, 0)),
        pl.BlockSpec((H, H), lambda i, t: (0, 0)),
        pl.BlockSpec((1, H), lambda i, t: (0, 0)),
        pl.BlockSpec((H, NCLS), lambda i, t: (0, 0)),
        pl.BlockSpec((1, NCLS), lambda i, t: (0, 0)),
    ],
    out_specs=pl.BlockSpec((_BB, NCLS), lambda i, t: (i, 0)),
    out_shape=jax.ShapeDtypeStruct((B, NCLS), jnp.float32),
    scratch_shapes=[
        pltpu.VMEM((_BB, H), jnp.float32),
        pltpu.VMEM((_BB, H), jnp.float32),
        pltpu.VMEM((_BB, 4 * H), jnp.float32),
    ],
    compiler_params=pltpu.CompilerParams(
        dimension_semantics=("arbitrary", "arbitrary"),
    ),
)


def kernel(behavior_ids, category_ids, product_service_ids, source_service_ids,
           quantity_values, sequence_length, behavior_table, category_table,
           product_table, source_table, qW, qb, W_ih, W_hh, b_ih, b_hh,
           cW1, cb1, cW2, cb2):
    del sequence_length

    left = jnp.concatenate([behavior_table, product_table], axis=0)
    right = jnp.concatenate([category_table, source_table], axis=0)
    z_l = jnp.zeros_like(left)
    z_r = jnp.zeros_like(right)
    table = jnp.concatenate([
        jnp.concatenate([left, z_l], axis=1),
        jnp.concatenate([z_r, right], axis=1),
    ], axis=0)
    idxa = jnp.stack(
        [behavior_ids.T.astype(jnp.int32),
         product_service_ids.T.astype(jnp.int32) + BV],
        axis=-1).reshape(_ROWS)
    idxb = jnp.stack(
        [category_ids.T.astype(jnp.int32) + (BV + PV),
         source_service_ids.T.astype(jnp.int32) + (BV + PV + CV)],
        axis=-1).reshape(_ROWS)

    emb = _gather_call()(table, idxa, idxb).reshape(L, B, 4 * ED)

    wq = W_ih[:, 4 * ED:]
    u = (wq @ qW[:, 0]).reshape(1, 4 * H)
    bias = (b_ih + b_hh + wq @ qb).reshape(1, 4 * H)
    q_t = quantity_values.T.reshape(L, B, 1)

    out = _lstm_call(
        emb, q_t,
        W_ih[:, :4 * ED].T, W_hh.T, bias, u,
        cW1.T, cb1.reshape(1, H), cW2.T, cb2.reshape(1, NCLS),
    )
    return out
